# baseline (speedup 1.0000x reference)
import jax
import jax.numpy as jnp
from jax import lax
from jax.experimental import pallas as pl
from jax.experimental.pallas import tpu as pltpu

_INC = 1024
_INN = 512
_OUTC = 2048
_OUTN = 64
_D = 4
_N = 32
_NB = 4

_CG = _OUTN
_CA = _OUTN + 1
_CM = _CA + _D


def _fgl_body(xa_ref, xb_ref, w2_ref, ctv_ref, pack_ref,
              y_ref, s_ref, wt_ref, wn_ref, sc_ref):
    step = pl.program_id(0)

    @pl.when(step == 0)
    def _init():
        iota = lax.broadcasted_iota(jnp.int32, (_OUTN, _INN), 1)
        st = jnp.zeros((_OUTN, _INN), jnp.float32)
        for d in range(_D):
            a_col = pack_ref[0:_OUTN, _CA + d:_CA + d + 1].astype(jnp.int32)
            m_col = pack_ref[0:_OUTN, _CM + d:_CM + d + 1]
            st = st + jnp.where(iota == a_col, m_col, 0.0)
        s_ref[...] = st.T
        wt_ref[...] = w2_ref[...].T
        v = ctv_ref[...]
        ctg_col = pack_ref[:, _CG:_CG + 1]
        scale = ctg_col * jax.lax.rsqrt(jnp.sum(v * v, axis=1, keepdims=True))
        wn_ref[...] = v * scale

    s = s_ref[...]
    wt = wt_ref[...]
    t_a = jnp.dot(xa_ref[...].reshape(_NB // 2 * _INC, _INN), s,
                  preferred_element_type=jnp.float32)
    t_b = jnp.dot(xb_ref[...].reshape(_NB // 2 * _INC, _INN), s,
                  preferred_element_type=jnp.float32)
    for q in range(_NB // 2):
        sc_ref[:, _OUTN * q:_OUTN * (q + 1)] = t_a[_INC * q:_INC * (q + 1), :] * wt
    for q in range(_NB // 2):
        qq = q + _NB // 2
        sc_ref[:, _OUTN * qq:_OUTN * (qq + 1)] = t_b[_INC * q:_INC * (q + 1), :] * wt

    m2 = jnp.dot(wn_ref[...], sc_ref[...], preferred_element_type=jnp.float32)
    badd = pack_ref[:, :_OUTN]
    for q in range(_NB):
        y_ref[:, _OUTN * q:_OUTN * (q + 1)] = m2[:, _OUTN * q:_OUTN * (q + 1)] + badd


def kernel(x, A, mask, weight, ct_v, ct_g, ct_b, bias):
    w2 = weight.reshape(_OUTN, _INC)
    small = jnp.concatenate([A.astype(jnp.float32), mask.reshape(_OUTN, _D)], axis=1)
    smallp = jnp.pad(small, ((0, _OUTC - _OUTN), (0, 0)))
    pack = jnp.concatenate(
        [bias + ct_b[:, None], ct_g[:, None], smallp,
         jnp.zeros((_OUTC, 128 - _OUTN - 1 - 2 * _D), jnp.float32)], axis=1)

    halves = []
    nh = _N // 2
    for h in range(2):
        base = h * (nh // _NB)
        ay = pl.pallas_call(
            _fgl_body,
            grid=(nh // _NB,),
            in_specs=[
                pl.BlockSpec((_NB // 2, _INC, _INN),
                             lambda n, b=base: (2 * (n + b), 0, 0)),
                pl.BlockSpec((_NB // 2, _INC, _INN),
                             lambda n, b=base: (2 * (n + b) + 1, 0, 0)),
                pl.BlockSpec((_OUTN, _INC), lambda n: (0, 0)),
                pl.BlockSpec((_OUTC, _INC), lambda n: (0, 0)),
                pl.BlockSpec((_OUTC, 128), lambda n: (0, 0)),
            ],
            out_specs=pl.BlockSpec((_OUTC, _NB * _OUTN), lambda n: (0, n)),
            out_shape=jax.ShapeDtypeStruct((_OUTC, nh * _OUTN), jnp.float32),
            scratch_shapes=[
                pltpu.VMEM((_INN, _OUTN), jnp.float32),
                pltpu.VMEM((_INC, _OUTN), jnp.float32),
                pltpu.VMEM((_OUTC, _INC), jnp.float32),
                pltpu.VMEM((_INC, _NB * _OUTN), jnp.float32),
            ],
        )(x, x, w2, ct_v, pack)
        halves.append(ay.reshape(_OUTC, nh, _OUTN).transpose(1, 0, 2))
    return jnp.concatenate(halves, axis=0)


# speedup vs baseline: 1.3838x; 1.3838x over previous
import jax
import jax.numpy as jnp
from jax import lax
from jax.experimental import pallas as pl
from jax.experimental.pallas import tpu as pltpu

_INC = 1024
_INN = 512
_OUTC = 2048
_OUTN = 64
_D = 4
_N = 32
_NB = 4

_CG = _OUTN
_CA = _OUTN + 1
_CM = _CA + _D


def _fgl_body(xa_ref, xb_ref, w2_ref, ctv_ref, pack_ref,
              y_ref, s_ref, wt_ref, wn_ref, sc_ref):
    step = pl.program_id(0)

    @pl.when(step == 0)
    def _init():
        iota = lax.broadcasted_iota(jnp.int32, (_OUTN, _INN), 1)
        st = jnp.zeros((_OUTN, _INN), jnp.float32)
        for d in range(_D):
            a_col = pack_ref[0:_OUTN, _CA + d:_CA + d + 1].astype(jnp.int32)
            m_col = pack_ref[0:_OUTN, _CM + d:_CM + d + 1]
            st = st + jnp.where(iota == a_col, m_col, 0.0)
        s_ref[...] = st.T
        wt_ref[...] = w2_ref[...].T
        v = ctv_ref[...]
        ctg_col = pack_ref[:, _CG:_CG + 1]
        scale = ctg_col * jax.lax.rsqrt(jnp.sum(v * v, axis=1, keepdims=True))
        wn_ref[...] = v * scale

    s = s_ref[...]
    wt = wt_ref[...]
    t_a = jnp.dot(xa_ref[...].reshape(_NB // 2 * _INC, _INN), s,
                  preferred_element_type=jnp.float32)
    t_b = jnp.dot(xb_ref[...].reshape(_NB // 2 * _INC, _INN), s,
                  preferred_element_type=jnp.float32)
    for q in range(_NB // 2):
        sc_ref[:, _OUTN * q:_OUTN * (q + 1)] = t_a[_INC * q:_INC * (q + 1), :] * wt
    for q in range(_NB // 2):
        qq = q + _NB // 2
        sc_ref[:, _OUTN * qq:_OUTN * (qq + 1)] = t_b[_INC * q:_INC * (q + 1), :] * wt

    m2 = jnp.dot(wn_ref[...], sc_ref[...], preferred_element_type=jnp.float32)
    badd = pack_ref[:, :_OUTN]
    for q in range(_NB):
        y_ref[:, _OUTN * q:_OUTN * (q + 1)] = (
            m2[:, _OUTN * q:_OUTN * (q + 1)] + badd).astype(jnp.bfloat16)


def kernel(x, A, mask, weight, ct_v, ct_g, ct_b, bias):
    w2 = weight.reshape(_OUTN, _INC)
    small = jnp.concatenate([A.astype(jnp.float32), mask.reshape(_OUTN, _D)], axis=1)
    smallp = jnp.pad(small, ((0, _OUTC - _OUTN), (0, 0)))
    pack = jnp.concatenate(
        [bias + ct_b[:, None], ct_g[:, None], smallp,
         jnp.zeros((_OUTC, 128 - _OUTN - 1 - 2 * _D), jnp.float32)], axis=1)

    ay = pl.pallas_call(
        _fgl_body,
        grid=(_N // _NB,),
        in_specs=[
            pl.BlockSpec((_NB // 2, _INC, _INN), lambda n: (2 * n, 0, 0)),
            pl.BlockSpec((_NB // 2, _INC, _INN), lambda n: (2 * n + 1, 0, 0)),
            pl.BlockSpec((_OUTN, _INC), lambda n: (0, 0)),
            pl.BlockSpec((_OUTC, _INC), lambda n: (0, 0)),
            pl.BlockSpec((_OUTC, 128), lambda n: (0, 0)),
        ],
        out_specs=pl.BlockSpec((_OUTC, _NB * _OUTN), lambda n: (0, n)),
        out_shape=jax.ShapeDtypeStruct((_OUTC, _N * _OUTN), jnp.bfloat16),
        scratch_shapes=[
            pltpu.VMEM((_INN, _OUTN), jnp.float32),
            pltpu.VMEM((_INC, _OUTN), jnp.float32),
            pltpu.VMEM((_OUTC, _INC), jnp.float32),
            pltpu.VMEM((_INC, _NB * _OUTN), jnp.float32),
        ],
    )(x, x, w2, ct_v, pack)
    return ay.reshape(_OUTC, _N, _OUTN).transpose(1, 0, 2).astype(jnp.float32)


# speedup vs baseline: 1.3860x; 1.0016x over previous
import jax
import jax.numpy as jnp
from jax import lax
from jax.experimental import pallas as pl
from jax.experimental.pallas import tpu as pltpu

_INC = 1024
_INN = 512
_OUTC = 2048
_OUTN = 64
_D = 4
_N = 32
_NB = 4

_CG = _OUTN
_CA = _OUTN + 1
_CM = _CA + _D


def _fgl_body(xa_ref, xb_ref, w2_ref, ctv_ref, pack_ref,
              y_ref, s_ref, wt_ref, wn_ref, sc_ref):
    step = pl.program_id(0)

    @pl.when(step == 0)
    def _init():
        iota = lax.broadcasted_iota(jnp.int32, (_OUTN, _INN), 1)
        st = jnp.zeros((_OUTN, _INN), jnp.float32)
        for d in range(_D):
            a_col = pack_ref[0:_OUTN, _CA + d:_CA + d + 1].astype(jnp.int32)
            m_col = pack_ref[0:_OUTN, _CM + d:_CM + d + 1]
            st = st + jnp.where(iota == a_col, m_col, 0.0)
        s_ref[...] = st.T
        wt_ref[...] = w2_ref[...].T
        v = ctv_ref[...]
        ctg_col = pack_ref[:, _CG:_CG + 1]
        scale = ctg_col * jax.lax.rsqrt(jnp.sum(v * v, axis=1, keepdims=True))
        wn_ref[...] = (v * scale).astype(jnp.bfloat16)

    s = s_ref[...]
    wt = wt_ref[...]
    t_a = jnp.dot(xa_ref[...].reshape(_NB // 2 * _INC, _INN), s,
                  preferred_element_type=jnp.float32)
    t_b = jnp.dot(xb_ref[...].reshape(_NB // 2 * _INC, _INN), s,
                  preferred_element_type=jnp.float32)
    for q in range(_NB // 2):
        sc_ref[:, _OUTN * q:_OUTN * (q + 1)] = (
            t_a[_INC * q:_INC * (q + 1), :] * wt).astype(jnp.bfloat16)
    for q in range(_NB // 2):
        qq = q + _NB // 2
        sc_ref[:, _OUTN * qq:_OUTN * (qq + 1)] = (
            t_b[_INC * q:_INC * (q + 1), :] * wt).astype(jnp.bfloat16)

    m2 = jnp.dot(wn_ref[...], sc_ref[...], preferred_element_type=jnp.float32)
    badd = pack_ref[:, :_OUTN]
    for q in range(_NB):
        y_ref[:, _OUTN * q:_OUTN * (q + 1)] = (
            m2[:, _OUTN * q:_OUTN * (q + 1)] + badd).astype(jnp.bfloat16)


def kernel(x, A, mask, weight, ct_v, ct_g, ct_b, bias):
    w2 = weight.reshape(_OUTN, _INC)
    small = jnp.concatenate([A.astype(jnp.float32), mask.reshape(_OUTN, _D)], axis=1)
    smallp = jnp.pad(small, ((0, _OUTC - _OUTN), (0, 0)))
    pack = jnp.concatenate(
        [bias + ct_b[:, None], ct_g[:, None], smallp,
         jnp.zeros((_OUTC, 128 - _OUTN - 1 - 2 * _D), jnp.float32)], axis=1)

    ay = pl.pallas_call(
        _fgl_body,
        grid=(_N // _NB,),
        in_specs=[
            pl.BlockSpec((_NB // 2, _INC, _INN), lambda n: (2 * n, 0, 0)),
            pl.BlockSpec((_NB // 2, _INC, _INN), lambda n: (2 * n + 1, 0, 0)),
            pl.BlockSpec((_OUTN, _INC), lambda n: (0, 0)),
            pl.BlockSpec((_OUTC, _INC), lambda n: (0, 0)),
            pl.BlockSpec((_OUTC, 128), lambda n: (0, 0)),
        ],
        out_specs=pl.BlockSpec((_OUTC, _NB * _OUTN), lambda n: (0, n)),
        out_shape=jax.ShapeDtypeStruct((_OUTC, _N * _OUTN), jnp.bfloat16),
        scratch_shapes=[
            pltpu.VMEM((_INN, _OUTN), jnp.float32),
            pltpu.VMEM((_INC, _OUTN), jnp.float32),
            pltpu.VMEM((_OUTC, _INC), jnp.bfloat16),
            pltpu.VMEM((_INC, _NB * _OUTN), jnp.bfloat16),
        ],
    )(x, x, w2, ct_v, pack)
    return ay.reshape(_OUTC, _N, _OUTN).transpose(1, 0, 2).astype(jnp.float32)
